# Initial kernel scaffold; baseline (speedup 1.0000x reference)
import jax
import jax.numpy as jnp
from jax.experimental import pallas as pl


def kernel(x, mask):
    raise NotImplementedError("write your pallas kernel here")


# speedup vs baseline: 3.7495x; 3.7495x over previous
import jax
import jax.numpy as jnp
from jax.experimental import pallas as pl
from jax.experimental.pallas import tpu as pltpu

_B, _S, _D = 16, 2048, 1024
_S_BLK = 256


def _body(mask_ref, x_ref, o_ref):
    j = pl.program_id(1)
    m_row = mask_ref[0, 0, :]
    length = jnp.sum(m_row)
    m_blk = mask_ref[0, 0, pl.ds(j * _S_BLK, _S_BLK)]
    pos = jax.lax.broadcasted_iota(jnp.int32, (_S_BLK, 1), 0) + j * _S_BLK
    keep = (m_blk.reshape(_S_BLK, 1) != 0) & (pos < length)
    o_ref[0] = x_ref[0] * keep.astype(jnp.float32)


def kernel(x, mask):
    m3 = mask.astype(jnp.int32).reshape(_B, 1, _S)
    return pl.pallas_call(
        _body,
        grid=(_B, _S // _S_BLK),
        in_specs=[
            pl.BlockSpec((1, 1, _S), lambda b, j: (b, 0, 0)),
            pl.BlockSpec((1, _S_BLK, _D), lambda b, j: (b, j, 0)),
        ],
        out_specs=pl.BlockSpec((1, _S_BLK, _D), lambda b, j: (b, j, 0)),
        out_shape=jax.ShapeDtypeStruct((_B, _S, _D), jnp.float32),
    )(m3, x)


# speedup vs baseline: 5.3623x; 1.4301x over previous
import jax
import jax.numpy as jnp
from jax.experimental import pallas as pl
from jax.experimental.pallas import tpu as pltpu

_B, _S, _D = 16, 2048, 1024
_S_BLK = 512


def _body(mask_ref, x_ref, o_ref):
    j = pl.program_id(1)
    m_row = mask_ref[0, 0, :]
    length = jnp.sum(m_row)
    m_blk = mask_ref[0, 0, pl.ds(j * _S_BLK, _S_BLK)]
    pos = jax.lax.broadcasted_iota(jnp.int32, (_S_BLK, 1), 0) + j * _S_BLK
    keep = (m_blk.reshape(_S_BLK, 1) != 0) & (pos < length)
    o_ref[0] = x_ref[0] * keep.astype(jnp.float32)


def kernel(x, mask):
    m3 = mask.astype(jnp.int32).reshape(_B, 1, _S)
    return pl.pallas_call(
        _body,
        grid=(_B, _S // _S_BLK),
        in_specs=[
            pl.BlockSpec((1, 1, _S), lambda b, j: (b, 0, 0)),
            pl.BlockSpec((1, _S_BLK, _D), lambda b, j: (b, j, 0)),
        ],
        out_specs=pl.BlockSpec((1, _S_BLK, _D), lambda b, j: (b, j, 0)),
        out_shape=jax.ShapeDtypeStruct((_B, _S, _D), jnp.float32),
    )(m3, x)


# speedup vs baseline: 5.9582x; 1.1111x over previous
import jax
import jax.numpy as jnp
from jax.experimental import pallas as pl
from jax.experimental.pallas import tpu as pltpu

_B, _S, _D = 16, 2048, 1024
_S_BLK = 1024


def _body(mask_ref, x_ref, o_ref):
    j = pl.program_id(1)
    m_row = mask_ref[0, 0, :]
    length = jnp.sum(m_row)
    m_blk = mask_ref[0, 0, pl.ds(j * _S_BLK, _S_BLK)]
    pos = jax.lax.broadcasted_iota(jnp.int32, (_S_BLK, 1), 0) + j * _S_BLK
    keep = (m_blk.reshape(_S_BLK, 1) != 0) & (pos < length)
    o_ref[0] = x_ref[0] * keep.astype(jnp.float32)


def kernel(x, mask):
    m3 = mask.astype(jnp.int32).reshape(_B, 1, _S)
    return pl.pallas_call(
        _body,
        grid=(_B, _S // _S_BLK),
        in_specs=[
            pl.BlockSpec((1, 1, _S), lambda b, j: (b, 0, 0)),
            pl.BlockSpec((1, _S_BLK, _D), lambda b, j: (b, j, 0)),
        ],
        out_specs=pl.BlockSpec((1, _S_BLK, _D), lambda b, j: (b, j, 0)),
        out_shape=jax.ShapeDtypeStruct((_B, _S, _D), jnp.float32),
    )(m3, x)


# speedup vs baseline: 6.1075x; 1.0251x over previous
import jax
import jax.numpy as jnp
from jax.experimental import pallas as pl
from jax.experimental.pallas import tpu as pltpu

_B, _S, _D = 16, 2048, 1024
_S_BLK = 2048


def _body(mask_ref, x_ref, o_ref):
    j = pl.program_id(1)
    m_row = mask_ref[0, 0, :]
    length = jnp.sum(m_row)
    m_blk = mask_ref[0, 0, pl.ds(j * _S_BLK, _S_BLK)]
    pos = jax.lax.broadcasted_iota(jnp.int32, (_S_BLK, 1), 0) + j * _S_BLK
    keep = (m_blk.reshape(_S_BLK, 1) != 0) & (pos < length)
    o_ref[0] = x_ref[0] * keep.astype(jnp.float32)


def kernel(x, mask):
    m3 = mask.astype(jnp.int32).reshape(_B, 1, _S)
    return pl.pallas_call(
        _body,
        grid=(_B, _S // _S_BLK),
        in_specs=[
            pl.BlockSpec((1, 1, _S), lambda b, j: (b, 0, 0)),
            pl.BlockSpec((1, _S_BLK, _D), lambda b, j: (b, j, 0)),
        ],
        out_specs=pl.BlockSpec((1, _S_BLK, _D), lambda b, j: (b, j, 0)),
        out_shape=jax.ShapeDtypeStruct((_B, _S, _D), jnp.float32),
    )(m3, x)
